# baseline (device time: 68252 ns/iter reference)
import jax
import jax.numpy as jnp
from jax import lax
from jax.experimental import pallas as pl
from jax.experimental.pallas import tpu as pltpu

N_DEV = 32
M_BLK = 128
GRP = 4
K_GRP = GRP * M_BLK
N_GROUPS = N_DEV // GRP
N_WBUF = 2


def _body(x_ref, w_ref, out_ref, xg, wbuf, avec, acomm,
          wsems, send_sems, recv_sems, asend_sems, arecv_sems):
    me = lax.axis_index("i")

    def issue_w_group(g):
        buf = g % N_WBUF
        cps = []
        for t in range(GRP):
            j = lax.rem(me + (g * GRP + t), N_DEV)
            cp = pltpu.make_async_copy(
                w_ref.at[pl.ds(j * M_BLK, M_BLK), :],
                wbuf.at[buf, pl.ds(t * M_BLK, M_BLK), :],
                wsems.at[buf, t],
            )
            cp.start()
            cps.append(cp)
        return cps

    wdma = {g: issue_w_group(g) for g in range(N_WBUF)}

    xg[:, pl.ds(0, M_BLK)] = x_ref[pl.ds(me * M_BLK, M_BLK), :]

    barrier = pltpu.get_barrier_semaphore()
    for h in range(1, N_DEV):
        tgt = lax.rem(me + h, N_DEV)
        pl.semaphore_signal(
            barrier, 1, device_id=(tgt,), device_id_type=pl.DeviceIdType.MESH
        )
    pl.semaphore_wait(barrier, N_DEV - 1)

    sends = []
    for s in range(1, N_DEV):
        tgt = lax.rem(me - s + N_DEV, N_DEV)
        rdma = pltpu.make_async_remote_copy(
            src_ref=x_ref.at[pl.ds(tgt * M_BLK, M_BLK), :],
            dst_ref=xg.at[:, pl.ds(s * M_BLK, M_BLK)],
            send_sem=send_sems.at[s],
            recv_sem=recv_sems.at[s],
            device_id=(tgt,),
            device_id_type=pl.DeviceIdType.MESH,
        )
        rdma.start()
        sends.append(rdma)

    for g in range(N_GROUPS):
        for cp in wdma[g]:
            cp.wait()
        for t in range(GRP):
            s = g * GRP + t
            if s == 0:
                continue
            recv = pltpu.make_async_remote_copy(
                src_ref=x_ref.at[pl.ds(0, M_BLK), :],
                dst_ref=xg.at[:, pl.ds(s * M_BLK, M_BLK)],
                send_sem=send_sems.at[s],
                recv_sem=recv_sems.at[s],
                device_id=(me,),
                device_id_type=pl.DeviceIdType.MESH,
            )
            recv.wait_recv()
        contrib = jnp.dot(
            xg[:, pl.ds(g * K_GRP, K_GRP)],
            wbuf[g % N_WBUF],
            preferred_element_type=jnp.float32,
        )
        if g == 0:
            out_ref[...] = contrib
        else:
            out_ref[...] += contrib
        if g + N_WBUF < N_GROUPS:
            wdma[g + N_WBUF] = issue_w_group(g + N_WBUF)

    y = jnp.maximum(out_ref[...], 0.0)
    out_ref[...] = y
    avec[...] = jnp.full((1, 128), jnp.max(y), jnp.float32)

    asends = []
    for h in range(1, N_DEV):
        tgt = lax.rem(me + h, N_DEV)
        rdma = pltpu.make_async_remote_copy(
            src_ref=avec,
            dst_ref=acomm.at[pl.ds(me, 1), :],
            send_sem=asend_sems.at[tgt],
            recv_sem=arecv_sems.at[me],
            device_id=(tgt,),
            device_id_type=pl.DeviceIdType.MESH,
        )
        rdma.start()
        asends.append(rdma)
    acomm[pl.ds(me, 1), :] = avec[...]

    for rdma in sends:
        rdma.wait_send()

    for h in range(1, N_DEV):
        src = lax.rem(me + h, N_DEV)
        recv = pltpu.make_async_remote_copy(
            src_ref=avec,
            dst_ref=acomm.at[pl.ds(src, 1), :],
            send_sem=asend_sems.at[src],
            recv_sem=arecv_sems.at[src],
            device_id=(src,),
            device_id_type=pl.DeviceIdType.MESH,
        )
        recv.wait_recv()

    scale = jnp.max(acomm[...]) / 127.0
    q = jnp.clip(jnp.round(out_ref[...] / scale), -127.0, 127.0)
    out_ref[...] = q * scale

    for rdma in asends:
        rdma.wait_send()


def kernel(x, w_mat):
    k, _ = x.shape
    _, n = w_mat.shape

    return pl.pallas_call(
        _body,
        out_shape=jax.ShapeDtypeStruct((M_BLK, n), jnp.float32),
        in_specs=[
            pl.BlockSpec(memory_space=pltpu.VMEM),
            pl.BlockSpec(memory_space=pltpu.MemorySpace.HBM),
        ],
        out_specs=pl.BlockSpec(memory_space=pltpu.VMEM),
        scratch_shapes=[
            pltpu.VMEM((M_BLK, k), jnp.float32),
            pltpu.VMEM((N_WBUF, K_GRP, n), jnp.float32),
            pltpu.VMEM((1, 128), jnp.float32),
            pltpu.VMEM((N_DEV, 128), jnp.float32),
            pltpu.SemaphoreType.DMA((N_WBUF, GRP)),
            pltpu.SemaphoreType.DMA((N_DEV,)),
            pltpu.SemaphoreType.DMA((N_DEV,)),
            pltpu.SemaphoreType.DMA((N_DEV,)),
            pltpu.SemaphoreType.DMA((N_DEV,)),
        ],
        compiler_params=pltpu.CompilerParams(
            collective_id=0, vmem_limit_bytes=56 * 1024 * 1024
        ),
    )(x, w_mat)


# device time: 63329 ns/iter; 1.0777x vs baseline; 1.0777x over previous
import jax
import jax.numpy as jnp
from jax import lax
from jax.experimental import pallas as pl
from jax.experimental.pallas import tpu as pltpu

N_DEV = 32
M_BLK = 128
GRP = 4
K_GRP = GRP * M_BLK
N_GROUPS = N_DEV // GRP
N_WBUF = 3


def _body(x_ref, w_ref, out_ref, acc, xg, wbuf,
          wsems, osems, send_sems, recv_sems, vsems):
    me = lax.axis_index("i")
    n = out_ref.shape[1]
    n_half = n // 2

    def issue_w_group(g):
        buf = g % N_WBUF
        cps = []
        for t in range(GRP):
            j = lax.rem(me + (g * GRP + t), N_DEV)
            for u in range(2):
                cp = pltpu.make_async_copy(
                    w_ref.at[pl.ds(j * M_BLK, M_BLK), pl.ds(u * n_half, n_half)],
                    wbuf.at[buf, pl.ds(t * M_BLK, M_BLK), pl.ds(u * n_half, n_half)],
                    wsems.at[buf, t, u],
                )
                cp.start()
                cps.append(cp)
        return cps

    wdma = {g: issue_w_group(g) for g in range(N_WBUF)}

    xg[:, pl.ds(0, M_BLK)] = x_ref[pl.ds(me * M_BLK, M_BLK), :]

    barrier = pltpu.get_barrier_semaphore()
    for h in range(1, N_DEV):
        tgt = lax.rem(me + h, N_DEV)
        pl.semaphore_signal(
            barrier, 1, device_id=(tgt,), device_id_type=pl.DeviceIdType.MESH
        )
    pl.semaphore_wait(barrier, N_DEV - 1)

    sends = []
    for s in range(1, N_DEV):
        tgt = lax.rem(me - s + N_DEV, N_DEV)
        rdma = pltpu.make_async_remote_copy(
            src_ref=x_ref.at[pl.ds(tgt * M_BLK, M_BLK), :],
            dst_ref=xg.at[:, pl.ds(s * M_BLK, M_BLK)],
            send_sem=send_sems.at[s],
            recv_sem=recv_sems.at[s],
            device_id=(tgt,),
            device_id_type=pl.DeviceIdType.MESH,
        )
        rdma.start()
        sends.append(rdma)

    def wait_block(g, t):
        for cp in wdma[g][2 * t:2 * t + 2]:
            cp.wait()
        s = g * GRP + t
        if s == 0:
            return
        recv = pltpu.make_async_remote_copy(
            src_ref=x_ref.at[pl.ds(0, M_BLK), :],
            dst_ref=xg.at[:, pl.ds(s * M_BLK, M_BLK)],
            send_sem=send_sems.at[s],
            recv_sem=recv_sems.at[s],
            device_id=(me,),
            device_id_type=pl.DeviceIdType.MESH,
        )
        recv.wait_recv()

    for g in range(N_GROUPS):
        if g < N_GROUPS - 1:
            for t in range(GRP):
                wait_block(g, t)
            contrib = jnp.dot(
                xg[:, pl.ds(g * K_GRP, K_GRP)],
                wbuf[g % N_WBUF],
                preferred_element_type=jnp.float32,
            )
            if g == 0:
                acc[...] = contrib
            else:
                acc[...] += contrib
        else:
            buf = g % N_WBUF
            for t in range(3):
                wait_block(g, t)
            acc[...] += jnp.dot(
                xg[:, pl.ds(g * K_GRP, 3 * M_BLK)],
                wbuf[buf, pl.ds(0, 3 * M_BLK), :],
                preferred_element_type=jnp.float32,
            )
            wait_block(g, 3)
            final = acc[...] + jnp.dot(
                xg[:, pl.ds(g * K_GRP + 3 * M_BLK, M_BLK)],
                wbuf[buf, pl.ds(3 * M_BLK, M_BLK), :],
                preferred_element_type=jnp.float32,
            )
            acc[...] = final
            amax_local = jnp.maximum(jnp.max(final), 0.0)
        if g + N_WBUF < N_GROUPS:
            wdma[g + N_WBUF] = issue_w_group(g + N_WBUF)

    my_bits = lax.bitcast_convert_type(amax_local, jnp.int32)
    for h in range(1, N_DEV):
        tgt = lax.rem(me + h, N_DEV)
        pl.semaphore_signal(
            vsems.at[me], my_bits,
            device_id=(tgt,), device_id_type=pl.DeviceIdType.MESH,
        )

    for rdma in sends:
        rdma.wait_send()

    gbits = my_bits
    for h in range(1, N_DEV):
        src = lax.rem(me + h, N_DEV)
        pl.semaphore_wait(vsems.at[src], 1, decrement=False)
        v = pl.semaphore_read(vsems.at[src])
        gbits = jnp.maximum(gbits, v)
        pl.semaphore_wait(vsems.at[src], v)

    gmax = lax.bitcast_convert_type(gbits, jnp.float32)
    scale = gmax / 127.0
    inv = 127.0 / gmax
    ocps = []
    n_q = n // 4
    for u in range(4):
        sl = pl.ds(u * n_q, n_q)
        q = jnp.minimum(jnp.round(jnp.maximum(acc[:, sl], 0.0) * inv), 127.0)
        acc[:, sl] = q * scale
        ocp = pltpu.make_async_copy(acc.at[:, sl], out_ref.at[:, sl], osems.at[u])
        ocp.start()
        ocps.append(ocp)
    for ocp in ocps:
        ocp.wait()


def kernel(x, w_mat):
    k, _ = x.shape
    _, n = w_mat.shape

    return pl.pallas_call(
        _body,
        out_shape=jax.ShapeDtypeStruct((M_BLK, n), jnp.float32),
        in_specs=[
            pl.BlockSpec(memory_space=pltpu.VMEM),
            pl.BlockSpec(memory_space=pltpu.MemorySpace.HBM),
        ],
        out_specs=pl.BlockSpec(memory_space=pltpu.MemorySpace.HBM),
        scratch_shapes=[
            pltpu.VMEM((M_BLK, n), jnp.float32),
            pltpu.VMEM((M_BLK, k), jnp.float32),
            pltpu.VMEM((N_WBUF, K_GRP, n), jnp.float32),
            pltpu.SemaphoreType.DMA((N_WBUF, GRP, 2)),
            pltpu.SemaphoreType.DMA((4,)),
            pltpu.SemaphoreType.DMA((N_DEV,)),
            pltpu.SemaphoreType.DMA((N_DEV,)),
            pltpu.SemaphoreType.REGULAR((N_DEV,)),
        ],
        compiler_params=pltpu.CompilerParams(
            collective_id=0, vmem_limit_bytes=62 * 1024 * 1024
        ),
    )(x, w_mat)
